# baseline (speedup 1.0000x reference)
import functools

import jax
import jax.numpy as jnp
from jax import lax
from jax.experimental import pallas as pl
from jax.experimental.pallas import tpu as pltpu
from jax.experimental.pallas import tpu_sc as plsc

N_NODE = 10000
D = 128
NC = 2
NS = 16
NW = NC * NS
L = 16
CHUNK = 64
PAD_N = 10112
ROWS_PER_SUB = PAD_N // NS
CROWS = 80

CHUNKS_CITES = 157
CHUNKS_WRITES = 79


def _pad_edges(src, dst, n_chunks):
    epad = NW * n_chunks * CHUNK
    pad = epad - src.shape[0]
    src = jnp.concatenate([src, jnp.zeros((pad,), jnp.int32)])
    dst = jnp.concatenate([dst, jnp.full((pad,), N_NODE, jnp.int32)])
    return src, dst


def _pieces(total, step):
    out, k = [], 0
    while k < total:
        out.append((k, min(step, total - k)))
        k += step
    return out


def _sc_segment_sums(src_c, dst_c, src_w, dst_w, src_b, dst_b, xp, xa,
                     zfeat, ident):
    mesh = plsc.VectorSubcoreMesh(core_axis_name="c", subcore_axis_name="s")
    f32 = jnp.float32

    @functools.partial(
        pl.kernel,
        mesh=mesh,
        out_type=[
            jax.ShapeDtypeStruct((NC * PAD_N, D), f32),
            jax.ShapeDtypeStruct((NC * CROWS, D), f32),
            jax.ShapeDtypeStruct((NC * PAD_N, D), f32),
            jax.ShapeDtypeStruct((NC * CROWS, D), f32),
            jax.ShapeDtypeStruct((NC * PAD_N, D), f32),
            jax.ShapeDtypeStruct((NC * CROWS, D), f32),
        ],
        scratch_types=(
            [pltpu.VMEM((CHUNK,), jnp.int32)] * 16
            + [pltpu.VMEM((CHUNK, D), f32)] * 4
            + [
                pltpu.VMEM_SHARED((PAD_N, D), f32),
                pltpu.VMEM_SHARED((CROWS, D), f32),
            ]
            + [pltpu.SemaphoreType.DMA] * 12
        ),
    )
    def sc_kernel(src_c_h, dst_c_h, src_w_h, dst_w_h, src_b_h, dst_b_h,
                  xp_h, xa_h, zfeat_h, ident_h,
                  o_sum_c, o_cnt_c, o_sum_w, o_cnt_w, o_sum_b, o_cnt_b,
                  *scratch):
        idxbufs = scratch[0:16]
        rowbufs = scratch[16:20]
        accum, cnt_acc = scratch[20], scratch[21]
        sems = scratch[22:34]
        isl = [idxbufs[4 * i:4 * i + 4] + (sems[i],) for i in range(4)]
        rsl = [
            (rowbufs[0], rowbufs[2], sems[4], sems[6], sems[8], sems[10]),
            (rowbufs[1], rowbufs[3], sems[5], sems[7], sems[9], sems[11]),
        ]
        rows_f0 = rowbufs[0]
        c = lax.axis_index("c")
        s = lax.axis_index("s")
        wid = c * NS + s
        r0 = s * ROWS_PER_SUB
        fpieces = _pieces(ROWS_PER_SUB, CHUNK)

        def idx_prefetch(src_h, dst_h, off, i):
            pltpu.async_copy(src_h.at[pl.ds(off, CHUNK)], isl[i][0],
                             isl[i][4])
            pltpu.async_copy(dst_h.at[pl.ds(off, CHUNK)], isl[i][1],
                             isl[i][4])

        def idx_wait(src_h, dst_h, i):
            pltpu.make_async_copy(src_h.at[pl.ds(0, CHUNK)], isl[i][0],
                                  isl[i][4]).wait()
            pltpu.make_async_copy(dst_h.at[pl.ds(0, CHUNK)], isl[i][1],
                                  isl[i][4]).wait()

        def compute_derived(i):
            didx, drow, dlane = isl[i][1], isl[i][2], isl[i][3]
            for t in range(CHUNK // L):
                dv = didx[pl.ds(t * L, L)]
                drow[pl.ds(t * L, L)] = lax.shift_right_logical(dv, 7)
                dlane[pl.ds(t * L, L)] = lax.bitwise_and(dv, 127)

        def gather_issue(x_h, i, r):
            pltpu.async_copy(x_h.at[isl[i][0]], rsl[r][0], rsl[r][2])
            pltpu.async_copy(ident_h.at[isl[i][3]], rsl[r][1], rsl[r][3])

        def gather_wait(r):
            pltpu.make_async_copy(zfeat_h, rsl[r][0], rsl[r][2]).wait()
            pltpu.make_async_copy(zfeat_h, rsl[r][1], rsl[r][3]).wait()

        def scatter_issue(i, r):
            pltpu.async_copy(rsl[r][0], accum.at[isl[i][1]], rsl[r][4],
                             add=True)
            pltpu.async_copy(rsl[r][1], cnt_acc.at[isl[i][2]], rsl[r][5],
                             add=True)

        def scatter_wait(r):
            pltpu.make_async_copy(zfeat_h, rsl[r][0], rsl[r][4]).wait()
            pltpu.make_async_copy(zfeat_h, rsl[r][1], rsl[r][5]).wait()

        def run_relation(src_h, dst_h, x_h, out_sum, out_cnt, n):
            pltpu.sync_copy(zfeat_h, rows_f0)
            for (k, sz) in fpieces:
                pltpu.sync_copy(rows_f0.at[pl.ds(0, sz)],
                                accum.at[pl.ds(r0 + k, sz)])

            @pl.when(s < CROWS // 8)
            def _():
                pltpu.sync_copy(rows_f0.at[pl.ds(0, 8)],
                                cnt_acc.at[pl.ds(s * 8, 8)])

            plsc.subcore_barrier()
            base = wid * (n * CHUNK)

            pltpu.sync_copy(src_h.at[pl.ds(base, CHUNK)], isl[0][0])
            pltpu.sync_copy(dst_h.at[pl.ds(base, CHUNK)], isl[0][1])
            compute_derived(0)
            gather_issue(x_h, 0, 0)
            idx_prefetch(src_h, dst_h,
                         pl.multiple_of(base + CHUNK, 8), 1)

            def iter_body(g, i, r, gg0):
                i1 = (i + 1) % 4
                i2 = (i + 2) % 4
                r1 = 1 - r

                @pl.when(g < n)
                def _():
                    gather_wait(r)
                    scatter_issue(i, r)

                    @pl.when(g + 1 < n)
                    def _():
                        idx_wait(src_h, dst_h, i1)
                        compute_derived(i1)
                        if gg0 is None:
                            scatter_wait(r1)
                        else:
                            @pl.when(gg0 > 0)
                            def _():
                                scatter_wait(r1)
                        gather_issue(x_h, i1, r1)

                    @pl.when(g + 2 < n)
                    def _():
                        off = pl.multiple_of(base + (g + 2) * CHUNK, 8)
                        idx_prefetch(src_h, dst_h, off, i2)

            def quad_body(qq, carry):
                g = 4 * qq
                iter_body(g, 0, 0, qq)
                iter_body(g + 1, 1, 1, None)
                iter_body(g + 2, 2, 0, None)
                iter_body(g + 3, 3, 1, None)
                return carry

            lax.fori_loop(0, (n + 3) // 4, quad_body, 0)
            scatter_wait((n - 1) % 2)
            scatter_wait(n % 2)
            plsc.subcore_barrier()
            for (k, sz) in fpieces:
                pltpu.sync_copy(accum.at[pl.ds(r0 + k, sz)],
                                rows_f0.at[pl.ds(0, sz)])
                pltpu.sync_copy(rows_f0.at[pl.ds(0, sz)],
                                out_sum.at[pl.ds(c * PAD_N + r0 + k, sz)])

            @pl.when(s < CROWS // 8)
            def _():
                pltpu.sync_copy(cnt_acc.at[pl.ds(s * 8, 8)],
                                rowbufs[2].at[pl.ds(0, 8)])
                pltpu.sync_copy(rowbufs[2].at[pl.ds(0, 8)],
                                out_cnt.at[pl.ds(c * CROWS + s * 8, 8)])

            plsc.subcore_barrier()

        run_relation(src_c_h, dst_c_h, xp_h, o_sum_c, o_cnt_c, CHUNKS_CITES)
        run_relation(src_w_h, dst_w_h, xa_h, o_sum_w, o_cnt_w, CHUNKS_WRITES)
        run_relation(src_b_h, dst_b_h, xp_h, o_sum_b, o_cnt_b, CHUNKS_WRITES)

    return sc_kernel(src_c, dst_c, src_w, dst_w, src_b, dst_b, xp, xa,
                     zfeat, ident)


def _tc_epilogue_body(sc0_ref, sc1_ref, sw0_ref, sw1_ref, sb0_ref, sb1_ref,
                      cc0_ref, cc1_ref, cw0_ref, cw1_ref, cb0_ref, cb1_ref,
                      xp_ref, xa_ref,
                      wlc_ref, wrc_ref, wlw_ref, wrw_ref, wlb_ref, wrb_ref,
                      blc_ref, brc_ref, blw_ref, brw_ref, blb_ref, brb_ref,
                      outp_ref, outa_ref):
    f32 = jnp.float32

    def mean(p0_ref, p1_ref, c0_ref, c1_ref):
        cnt = c0_ref[...] + c1_ref[...]
        return (p0_ref[...] + p1_ref[...]) / jnp.maximum(cnt, 1.0)

    mc = mean(sc0_ref, sc1_ref, cc0_ref, cc1_ref)
    mw = mean(sw0_ref, sw1_ref, cw0_ref, cw1_ref)
    mb = mean(sb0_ref, sb1_ref, cb0_ref, cb1_ref)
    outp_ref[...] = (
        jnp.dot(mc, wlc_ref[...], preferred_element_type=f32)
        + jnp.dot(mw, wlw_ref[...], preferred_element_type=f32)
        + jnp.dot(xp_ref[...], wrc_ref[...] + wrw_ref[...],
                  preferred_element_type=f32)
        + blc_ref[...] + brc_ref[...] + blw_ref[...] + brw_ref[...]
    )
    outa_ref[...] = (
        jnp.dot(mb, wlb_ref[...], preferred_element_type=f32)
        + jnp.dot(xa_ref[...], wrb_ref[...], preferred_element_type=f32)
        + blb_ref[...] + brb_ref[...]
    )


def _tc_epilogue(feats, cnts, xp, xa, ws, bs):
    f32 = jnp.float32
    bm = 1000
    grid = (N_NODE // bm,)
    feat = pl.BlockSpec((bm, D), lambda i: (i, 0))
    cnt = pl.BlockSpec((bm, 1), lambda i: (i, 0))
    wmat = pl.BlockSpec((D, D), lambda i: (0, 0))
    bvec = pl.BlockSpec((1, D), lambda i: (0, 0))
    sc0, sw0, sb0, sc1, sw1, sb1 = feats
    cc0, cw0, cb0, cc1, cw1, cb1 = cnts
    return pl.pallas_call(
        _tc_epilogue_body,
        grid=grid,
        in_specs=[feat, feat, feat, feat, feat, feat,
                  cnt, cnt, cnt, cnt, cnt, cnt, feat, feat,
                  wmat, wmat, wmat, wmat, wmat, wmat,
                  bvec, bvec, bvec, bvec, bvec, bvec],
        out_specs=[feat, feat],
        out_shape=[jax.ShapeDtypeStruct((N_NODE, D), f32),
                   jax.ShapeDtypeStruct((N_NODE, D), f32)],
    )(sc0, sc1, sw0, sw1, sb0, sb1, cc0, cc1, cw0, cw1, cb0, cb1,
      xp, xa, *ws, *bs)


def _unpack_counts(cnt_out):
    res = []
    for cpart in (cnt_out[:CROWS], cnt_out[CROWS:]):
        res.append(cpart.reshape(CROWS * D)[:N_NODE].reshape(N_NODE, 1))
    return res


def kernel(x_paper, x_author, edge_index_cites, edge_index_writes,
           edge_index_written_by,
           Wl_cites, bl_cites, Wr_cites, br_cites,
           Wl_writes, bl_writes, Wr_writes, br_writes,
           Wl_wb, bl_wb, Wr_wb, br_wb):
    i32 = jnp.int32
    f32 = jnp.float32
    ec = edge_index_cites.astype(i32)
    ew = edge_index_writes.astype(i32)
    eb = edge_index_written_by.astype(i32)
    src_c, dst_c = _pad_edges(ec[0], ec[1], CHUNKS_CITES)
    src_w, dst_w = _pad_edges(ew[0], ew[1], CHUNKS_WRITES)
    src_b, dst_b = _pad_edges(eb[0], eb[1], CHUNKS_WRITES)

    zfeat = jnp.zeros((CHUNK, D), f32)
    ident = jnp.eye(D, dtype=f32)

    sum_c, cnt_c, sum_w, cnt_w, sum_b, cnt_b = _sc_segment_sums(
        src_c, dst_c, src_w, dst_w, src_b, dst_b, x_paper, x_author,
        zfeat, ident)

    feats = (sum_c[:N_NODE], sum_w[:N_NODE], sum_b[:N_NODE],
             sum_c[PAD_N:PAD_N + N_NODE], sum_w[PAD_N:PAD_N + N_NODE],
             sum_b[PAD_N:PAD_N + N_NODE])
    cc0, cc1 = _unpack_counts(cnt_c)
    cw0, cw1 = _unpack_counts(cnt_w)
    cb0, cb1 = _unpack_counts(cnt_b)
    cnts = (cc0, cw0, cb0, cc1, cw1, cb1)
    ws = (Wl_cites, Wr_cites, Wl_writes, Wr_writes, Wl_wb, Wr_wb)
    bs = (bl_cites.reshape(1, D), br_cites.reshape(1, D),
          bl_writes.reshape(1, D), br_writes.reshape(1, D),
          bl_wb.reshape(1, D), br_wb.reshape(1, D))
    out_p, out_a = _tc_epilogue(feats, cnts, x_paper, x_author, ws, bs)
    return (out_p, out_a)


# speedup vs baseline: 1.1319x; 1.1319x over previous
import functools

import jax
import jax.numpy as jnp
from jax import lax
from jax.experimental import pallas as pl
from jax.experimental.pallas import tpu as pltpu
from jax.experimental.pallas import tpu_sc as plsc

N_NODE = 10000
D = 128
NC = 2
NS = 16
NW = NC * NS
L = 16
CHUNK = 80
PAD_N = 10112
ROWS_PER_SUB = PAD_N // NS
CROWS = 80

CHUNKS_CITES = 125
CHUNKS_WRITES = 63


def _pad_edges(src, dst, n_chunks):
    epad = NW * n_chunks * CHUNK
    pad = epad - src.shape[0]
    src = jnp.concatenate([src, jnp.zeros((pad,), jnp.int32)])
    dst = jnp.concatenate([dst, jnp.full((pad,), N_NODE, jnp.int32)])
    return src, dst


def _pieces(total, step):
    out, k = [], 0
    while k < total:
        out.append((k, min(step, total - k)))
        k += step
    return out


def _sc_segment_sums(src_c, dst_c, src_w, dst_w, src_b, dst_b, xp, xa,
                     zfeat, ident):
    mesh = plsc.VectorSubcoreMesh(core_axis_name="c", subcore_axis_name="s")
    f32 = jnp.float32

    @functools.partial(
        pl.kernel,
        mesh=mesh,
        out_type=[
            jax.ShapeDtypeStruct((NC * PAD_N, D), f32),
            jax.ShapeDtypeStruct((NC * CROWS, D), f32),
            jax.ShapeDtypeStruct((NC * PAD_N, D), f32),
            jax.ShapeDtypeStruct((NC * CROWS, D), f32),
            jax.ShapeDtypeStruct((NC * PAD_N, D), f32),
            jax.ShapeDtypeStruct((NC * CROWS, D), f32),
        ],
        scratch_types=(
            [pltpu.VMEM((CHUNK,), jnp.int32)] * 16
            + [pltpu.VMEM((CHUNK, D), f32)] * 4
            + [
                pltpu.VMEM_SHARED((PAD_N, D), f32),
                pltpu.VMEM_SHARED((CROWS, D), f32),
            ]
            + [pltpu.SemaphoreType.DMA] * 12
        ),
    )
    def sc_kernel(src_c_h, dst_c_h, src_w_h, dst_w_h, src_b_h, dst_b_h,
                  xp_h, xa_h, zfeat_h, ident_h,
                  o_sum_c, o_cnt_c, o_sum_w, o_cnt_w, o_sum_b, o_cnt_b,
                  *scratch):
        idxbufs = scratch[0:16]
        rowbufs = scratch[16:20]
        accum, cnt_acc = scratch[20], scratch[21]
        sems = scratch[22:34]
        isl = [idxbufs[4 * i:4 * i + 4] + (sems[i],) for i in range(4)]
        rsl = [
            (rowbufs[0], rowbufs[2], sems[4], sems[6], sems[8], sems[10]),
            (rowbufs[1], rowbufs[3], sems[5], sems[7], sems[9], sems[11]),
        ]
        rows_f0 = rowbufs[0]
        c = lax.axis_index("c")
        s = lax.axis_index("s")
        wid = c * NS + s
        r0 = s * ROWS_PER_SUB
        fpieces = _pieces(ROWS_PER_SUB, CHUNK)

        def idx_prefetch(src_h, dst_h, off, i):
            pltpu.async_copy(src_h.at[pl.ds(off, CHUNK)], isl[i][0],
                             isl[i][4])
            pltpu.async_copy(dst_h.at[pl.ds(off, CHUNK)], isl[i][1],
                             isl[i][4])

        def idx_wait(src_h, dst_h, i):
            pltpu.make_async_copy(src_h.at[pl.ds(0, CHUNK)], isl[i][0],
                                  isl[i][4]).wait()
            pltpu.make_async_copy(dst_h.at[pl.ds(0, CHUNK)], isl[i][1],
                                  isl[i][4]).wait()

        def compute_derived(i):
            didx, drow, dlane = isl[i][1], isl[i][2], isl[i][3]
            for t in range(CHUNK // L):
                dv = didx[pl.ds(t * L, L)]
                drow[pl.ds(t * L, L)] = lax.shift_right_logical(dv, 7)
                dlane[pl.ds(t * L, L)] = lax.bitwise_and(dv, 127)

        def gather_issue(x_h, i, r):
            pltpu.async_copy(x_h.at[isl[i][0]], rsl[r][0], rsl[r][2])
            pltpu.async_copy(ident_h.at[isl[i][3]], rsl[r][1], rsl[r][3])

        def gather_wait(r):
            pltpu.make_async_copy(zfeat_h, rsl[r][0], rsl[r][2]).wait()
            pltpu.make_async_copy(zfeat_h, rsl[r][1], rsl[r][3]).wait()

        def scatter_issue(i, r):
            pltpu.async_copy(rsl[r][0], accum.at[isl[i][1]], rsl[r][4],
                             add=True)
            pltpu.async_copy(rsl[r][1], cnt_acc.at[isl[i][2]], rsl[r][5],
                             add=True)

        def scatter_wait(r):
            pltpu.make_async_copy(zfeat_h, rsl[r][0], rsl[r][4]).wait()
            pltpu.make_async_copy(zfeat_h, rsl[r][1], rsl[r][5]).wait()

        def run_relation(src_h, dst_h, x_h, out_sum, out_cnt, n):
            pltpu.sync_copy(zfeat_h, rows_f0)
            for (k, sz) in fpieces:
                pltpu.sync_copy(rows_f0.at[pl.ds(0, sz)],
                                accum.at[pl.ds(r0 + k, sz)])

            @pl.when(s < CROWS // 8)
            def _():
                pltpu.sync_copy(rows_f0.at[pl.ds(0, 8)],
                                cnt_acc.at[pl.ds(s * 8, 8)])

            plsc.subcore_barrier()
            base = wid * (n * CHUNK)

            pltpu.sync_copy(src_h.at[pl.ds(base, CHUNK)], isl[0][0])
            pltpu.sync_copy(dst_h.at[pl.ds(base, CHUNK)], isl[0][1])
            compute_derived(0)
            gather_issue(x_h, 0, 0)
            idx_prefetch(src_h, dst_h,
                         pl.multiple_of(base + CHUNK, 8), 1)

            def iter_body(g, i, r, gg0):
                i1 = (i + 1) % 4
                i2 = (i + 2) % 4
                r1 = 1 - r

                @pl.when(g < n)
                def _():
                    gather_wait(r)
                    scatter_issue(i, r)

                    @pl.when(g + 1 < n)
                    def _():
                        idx_wait(src_h, dst_h, i1)
                        compute_derived(i1)
                        if gg0 is None:
                            scatter_wait(r1)
                        else:
                            @pl.when(gg0 > 0)
                            def _():
                                scatter_wait(r1)
                        gather_issue(x_h, i1, r1)

                    @pl.when(g + 2 < n)
                    def _():
                        off = pl.multiple_of(base + (g + 2) * CHUNK, 8)
                        idx_prefetch(src_h, dst_h, off, i2)

            def quad_body(qq, carry):
                g = 4 * qq
                iter_body(g, 0, 0, qq)
                iter_body(g + 1, 1, 1, None)
                iter_body(g + 2, 2, 0, None)
                iter_body(g + 3, 3, 1, None)
                return carry

            lax.fori_loop(0, (n + 3) // 4, quad_body, 0)
            scatter_wait((n - 1) % 2)
            scatter_wait(n % 2)
            plsc.subcore_barrier()
            for (k, sz) in fpieces:
                pltpu.sync_copy(accum.at[pl.ds(r0 + k, sz)],
                                rows_f0.at[pl.ds(0, sz)])
                pltpu.sync_copy(rows_f0.at[pl.ds(0, sz)],
                                out_sum.at[pl.ds(c * PAD_N + r0 + k, sz)])

            @pl.when(s < CROWS // 8)
            def _():
                pltpu.sync_copy(cnt_acc.at[pl.ds(s * 8, 8)],
                                rowbufs[2].at[pl.ds(0, 8)])
                pltpu.sync_copy(rowbufs[2].at[pl.ds(0, 8)],
                                out_cnt.at[pl.ds(c * CROWS + s * 8, 8)])

            plsc.subcore_barrier()

        run_relation(src_c_h, dst_c_h, xp_h, o_sum_c, o_cnt_c, CHUNKS_CITES)
        run_relation(src_w_h, dst_w_h, xa_h, o_sum_w, o_cnt_w, CHUNKS_WRITES)
        run_relation(src_b_h, dst_b_h, xp_h, o_sum_b, o_cnt_b, CHUNKS_WRITES)

    return sc_kernel(src_c, dst_c, src_w, dst_w, src_b, dst_b, xp, xa,
                     zfeat, ident)


def _tc_epilogue_body(sc0_ref, sc1_ref, sw0_ref, sw1_ref, sb0_ref, sb1_ref,
                      cc0_ref, cc1_ref, cw0_ref, cw1_ref, cb0_ref, cb1_ref,
                      xp_ref, xa_ref,
                      wlc_ref, wrc_ref, wlw_ref, wrw_ref, wlb_ref, wrb_ref,
                      blc_ref, brc_ref, blw_ref, brw_ref, blb_ref, brb_ref,
                      outp_ref, outa_ref):
    f32 = jnp.float32

    def mean(p0_ref, p1_ref, c0_ref, c1_ref):
        cnt = c0_ref[...] + c1_ref[...]
        return (p0_ref[...] + p1_ref[...]) / jnp.maximum(cnt, 1.0)

    mc = mean(sc0_ref, sc1_ref, cc0_ref, cc1_ref)
    mw = mean(sw0_ref, sw1_ref, cw0_ref, cw1_ref)
    mb = mean(sb0_ref, sb1_ref, cb0_ref, cb1_ref)
    outp_ref[...] = (
        jnp.dot(mc, wlc_ref[...], preferred_element_type=f32)
        + jnp.dot(mw, wlw_ref[...], preferred_element_type=f32)
        + jnp.dot(xp_ref[...], wrc_ref[...] + wrw_ref[...],
                  preferred_element_type=f32)
        + blc_ref[...] + brc_ref[...] + blw_ref[...] + brw_ref[...]
    )
    outa_ref[...] = (
        jnp.dot(mb, wlb_ref[...], preferred_element_type=f32)
        + jnp.dot(xa_ref[...], wrb_ref[...], preferred_element_type=f32)
        + blb_ref[...] + brb_ref[...]
    )


def _tc_epilogue(feats, cnts, xp, xa, ws, bs):
    f32 = jnp.float32
    bm = 1000
    grid = (N_NODE // bm,)
    feat = pl.BlockSpec((bm, D), lambda i: (i, 0))
    cnt = pl.BlockSpec((bm, 1), lambda i: (i, 0))
    wmat = pl.BlockSpec((D, D), lambda i: (0, 0))
    bvec = pl.BlockSpec((1, D), lambda i: (0, 0))
    sc0, sw0, sb0, sc1, sw1, sb1 = feats
    cc0, cw0, cb0, cc1, cw1, cb1 = cnts
    return pl.pallas_call(
        _tc_epilogue_body,
        grid=grid,
        in_specs=[feat, feat, feat, feat, feat, feat,
                  cnt, cnt, cnt, cnt, cnt, cnt, feat, feat,
                  wmat, wmat, wmat, wmat, wmat, wmat,
                  bvec, bvec, bvec, bvec, bvec, bvec],
        out_specs=[feat, feat],
        out_shape=[jax.ShapeDtypeStruct((N_NODE, D), f32),
                   jax.ShapeDtypeStruct((N_NODE, D), f32)],
    )(sc0, sc1, sw0, sw1, sb0, sb1, cc0, cc1, cw0, cw1, cb0, cb1,
      xp, xa, *ws, *bs)


def _unpack_counts(cnt_out):
    res = []
    for cpart in (cnt_out[:CROWS], cnt_out[CROWS:]):
        res.append(cpart.reshape(CROWS * D)[:N_NODE].reshape(N_NODE, 1))
    return res


def kernel(x_paper, x_author, edge_index_cites, edge_index_writes,
           edge_index_written_by,
           Wl_cites, bl_cites, Wr_cites, br_cites,
           Wl_writes, bl_writes, Wr_writes, br_writes,
           Wl_wb, bl_wb, Wr_wb, br_wb):
    i32 = jnp.int32
    f32 = jnp.float32
    ec = edge_index_cites.astype(i32)
    ew = edge_index_writes.astype(i32)
    eb = edge_index_written_by.astype(i32)
    src_c, dst_c = _pad_edges(ec[0], ec[1], CHUNKS_CITES)
    src_w, dst_w = _pad_edges(ew[0], ew[1], CHUNKS_WRITES)
    src_b, dst_b = _pad_edges(eb[0], eb[1], CHUNKS_WRITES)

    zfeat = jnp.zeros((CHUNK, D), f32)
    ident = jnp.eye(D, dtype=f32)

    sum_c, cnt_c, sum_w, cnt_w, sum_b, cnt_b = _sc_segment_sums(
        src_c, dst_c, src_w, dst_w, src_b, dst_b, x_paper, x_author,
        zfeat, ident)

    feats = (sum_c[:N_NODE], sum_w[:N_NODE], sum_b[:N_NODE],
             sum_c[PAD_N:PAD_N + N_NODE], sum_w[PAD_N:PAD_N + N_NODE],
             sum_b[PAD_N:PAD_N + N_NODE])
    cc0, cc1 = _unpack_counts(cnt_c)
    cw0, cw1 = _unpack_counts(cnt_w)
    cb0, cb1 = _unpack_counts(cnt_b)
    cnts = (cc0, cw0, cb0, cc1, cw1, cb1)
    ws = (Wl_cites, Wr_cites, Wl_writes, Wr_writes, Wl_wb, Wr_wb)
    bs = (bl_cites.reshape(1, D), br_cites.reshape(1, D),
          bl_writes.reshape(1, D), br_writes.reshape(1, D),
          bl_wb.reshape(1, D), br_wb.reshape(1, D))
    out_p, out_a = _tc_epilogue(feats, cnts, x_paper, x_author, ws, bs)
    return (out_p, out_a)


# speedup vs baseline: 1.2237x; 1.0811x over previous
import functools

import jax
import jax.numpy as jnp
from jax import lax
from jax.experimental import pallas as pl
from jax.experimental.pallas import tpu as pltpu
from jax.experimental.pallas import tpu_sc as plsc

N_NODE = 10000
D = 128
NC = 2
NS = 16
NW = NC * NS
L = 16
CHUNK = 80
PAD_N = 10112
ROWS_PER_SUB = PAD_N // NS
CROWS = 80

CHUNKS_CITES = 125
CHUNKS_WRITES = 63


def _pad_edges(src, dst, n_chunks):
    epad = NW * n_chunks * CHUNK
    pad = epad - src.shape[0]
    src = jnp.concatenate([src, jnp.zeros((pad,), jnp.int32)])
    dst = jnp.concatenate([dst, jnp.full((pad,), N_NODE, jnp.int32)])
    return src, dst


def _pieces(total, step):
    out, k = [], 0
    while k < total:
        out.append((k, min(step, total - k)))
        k += step
    return out


def _sc_segment_sums(src_c, dst_c, src_w, dst_w, src_b, dst_b, xp, xa,
                     zfeat, ident):
    mesh = plsc.VectorSubcoreMesh(core_axis_name="c", subcore_axis_name="s")
    f32 = jnp.float32

    @functools.partial(
        pl.kernel,
        mesh=mesh,
        out_type=[
            jax.ShapeDtypeStruct((NC * PAD_N, D), f32),
            jax.ShapeDtypeStruct((NC * CROWS, D), f32),
            jax.ShapeDtypeStruct((NC * PAD_N, D), f32),
            jax.ShapeDtypeStruct((NC * CROWS, D), f32),
            jax.ShapeDtypeStruct((NC * PAD_N, D), f32),
            jax.ShapeDtypeStruct((NC * CROWS, D), f32),
        ],
        scratch_types=(
            [pltpu.VMEM((CHUNK,), jnp.int32)] * 16
            + [pltpu.VMEM((CHUNK, D), f32)] * 4
            + [
                pltpu.VMEM_SHARED((PAD_N, D), f32),
                pltpu.VMEM_SHARED((CROWS, D), f32),
                pltpu.VMEM_SHARED((D, D), f32),
            ]
            + [pltpu.SemaphoreType.DMA] * 12
        ),
    )
    def sc_kernel(src_c_h, dst_c_h, src_w_h, dst_w_h, src_b_h, dst_b_h,
                  xp_h, xa_h, zfeat_h, ident_h,
                  o_sum_c, o_cnt_c, o_sum_w, o_cnt_w, o_sum_b, o_cnt_b,
                  *scratch):
        idxbufs = scratch[0:16]
        rowbufs = scratch[16:20]
        accum, cnt_acc, ident_s = scratch[20], scratch[21], scratch[22]
        sems = scratch[23:35]
        isl = [idxbufs[4 * i:4 * i + 4] + (sems[i],) for i in range(4)]
        rsl = [
            (rowbufs[0], rowbufs[2], sems[4], sems[6], sems[8], sems[10]),
            (rowbufs[1], rowbufs[3], sems[5], sems[7], sems[9], sems[11]),
        ]
        rows_f0 = rowbufs[0]
        c = lax.axis_index("c")
        s = lax.axis_index("s")
        wid = c * NS + s
        r0 = s * ROWS_PER_SUB
        fpieces = _pieces(ROWS_PER_SUB, CHUNK)

        def idx_prefetch(src_h, dst_h, off, i):
            pltpu.async_copy(src_h.at[pl.ds(off, CHUNK)], isl[i][0],
                             isl[i][4])
            pltpu.async_copy(dst_h.at[pl.ds(off, CHUNK)], isl[i][1],
                             isl[i][4])

        def idx_wait(src_h, dst_h, i):
            pltpu.make_async_copy(src_h.at[pl.ds(0, CHUNK)], isl[i][0],
                                  isl[i][4]).wait()
            pltpu.make_async_copy(dst_h.at[pl.ds(0, CHUNK)], isl[i][1],
                                  isl[i][4]).wait()

        def compute_derived(i):
            didx, drow, dlane = isl[i][1], isl[i][2], isl[i][3]
            for t in range(CHUNK // L):
                dv = didx[pl.ds(t * L, L)]
                drow[pl.ds(t * L, L)] = lax.shift_right_logical(dv, 7)
                dlane[pl.ds(t * L, L)] = lax.bitwise_and(dv, 127)

        def gather_issue(x_h, i, r):
            pltpu.async_copy(x_h.at[isl[i][0]], rsl[r][0], rsl[r][2])
            pltpu.async_copy(ident_s.at[isl[i][3]], rsl[r][1], rsl[r][3])

        def gather_wait(r):
            pltpu.make_async_copy(zfeat_h, rsl[r][0], rsl[r][2]).wait()
            pltpu.make_async_copy(zfeat_h, rsl[r][1], rsl[r][3]).wait()

        def scatter_issue(i, r):
            pltpu.async_copy(rsl[r][0], accum.at[isl[i][1]], rsl[r][4],
                             add=True)
            pltpu.async_copy(rsl[r][1], cnt_acc.at[isl[i][2]], rsl[r][5],
                             add=True)

        def scatter_wait(r):
            pltpu.make_async_copy(zfeat_h, rsl[r][0], rsl[r][4]).wait()
            pltpu.make_async_copy(zfeat_h, rsl[r][1], rsl[r][5]).wait()

        def run_relation(src_h, dst_h, x_h, out_sum, out_cnt, n):
            pltpu.sync_copy(zfeat_h, rows_f0)
            for (k, sz) in fpieces:
                pltpu.sync_copy(rows_f0.at[pl.ds(0, sz)],
                                accum.at[pl.ds(r0 + k, sz)])

            @pl.when(s < CROWS // 8)
            def _():
                pltpu.sync_copy(rows_f0.at[pl.ds(0, 8)],
                                cnt_acc.at[pl.ds(s * 8, 8)])

            plsc.subcore_barrier()
            base = wid * (n * CHUNK)

            pltpu.sync_copy(src_h.at[pl.ds(base, CHUNK)], isl[0][0])
            pltpu.sync_copy(dst_h.at[pl.ds(base, CHUNK)], isl[0][1])
            compute_derived(0)
            gather_issue(x_h, 0, 0)
            idx_prefetch(src_h, dst_h,
                         pl.multiple_of(base + CHUNK, 8), 1)

            def iter_body(g, i, r, gg0):
                i1 = (i + 1) % 4
                i2 = (i + 2) % 4
                r1 = 1 - r

                @pl.when(g < n)
                def _():
                    gather_wait(r)
                    scatter_issue(i, r)

                    @pl.when(g + 1 < n)
                    def _():
                        idx_wait(src_h, dst_h, i1)
                        compute_derived(i1)
                        if gg0 is None:
                            scatter_wait(r1)
                        else:
                            @pl.when(gg0 > 0)
                            def _():
                                scatter_wait(r1)
                        gather_issue(x_h, i1, r1)

                    @pl.when(g + 2 < n)
                    def _():
                        off = pl.multiple_of(base + (g + 2) * CHUNK, 8)
                        idx_prefetch(src_h, dst_h, off, i2)

            def quad_body(qq, carry):
                g = 4 * qq
                iter_body(g, 0, 0, qq)
                iter_body(g + 1, 1, 1, None)
                iter_body(g + 2, 2, 0, None)
                iter_body(g + 3, 3, 1, None)
                return carry

            lax.fori_loop(0, (n + 3) // 4, quad_body, 0)
            scatter_wait((n - 1) % 2)
            scatter_wait(n % 2)
            plsc.subcore_barrier()
            for (k, sz) in fpieces:
                pltpu.sync_copy(accum.at[pl.ds(r0 + k, sz)],
                                rows_f0.at[pl.ds(0, sz)])
                pltpu.sync_copy(rows_f0.at[pl.ds(0, sz)],
                                out_sum.at[pl.ds(c * PAD_N + r0 + k, sz)])

            @pl.when(s < CROWS // 8)
            def _():
                pltpu.sync_copy(cnt_acc.at[pl.ds(s * 8, 8)],
                                rowbufs[2].at[pl.ds(0, 8)])
                pltpu.sync_copy(rowbufs[2].at[pl.ds(0, 8)],
                                out_cnt.at[pl.ds(c * CROWS + s * 8, 8)])

            plsc.subcore_barrier()

        @pl.when(s == 0)
        def _():
            for k in (0, 64):
                pltpu.sync_copy(ident_h.at[pl.ds(k, 64)],
                                rows_f0.at[pl.ds(0, 64)])
                pltpu.sync_copy(rows_f0.at[pl.ds(0, 64)],
                                ident_s.at[pl.ds(k, 64)])

        run_relation(src_c_h, dst_c_h, xp_h, o_sum_c, o_cnt_c, CHUNKS_CITES)
        run_relation(src_w_h, dst_w_h, xa_h, o_sum_w, o_cnt_w, CHUNKS_WRITES)
        run_relation(src_b_h, dst_b_h, xp_h, o_sum_b, o_cnt_b, CHUNKS_WRITES)

    return sc_kernel(src_c, dst_c, src_w, dst_w, src_b, dst_b, xp, xa,
                     zfeat, ident)


def _tc_epilogue_body(sc0_ref, sc1_ref, sw0_ref, sw1_ref, sb0_ref, sb1_ref,
                      cc0_ref, cc1_ref, cw0_ref, cw1_ref, cb0_ref, cb1_ref,
                      xp_ref, xa_ref,
                      wlc_ref, wrc_ref, wlw_ref, wrw_ref, wlb_ref, wrb_ref,
                      blc_ref, brc_ref, blw_ref, brw_ref, blb_ref, brb_ref,
                      outp_ref, outa_ref):
    f32 = jnp.float32

    def mean(p0_ref, p1_ref, c0_ref, c1_ref):
        cnt = c0_ref[...] + c1_ref[...]
        return (p0_ref[...] + p1_ref[...]) / jnp.maximum(cnt, 1.0)

    mc = mean(sc0_ref, sc1_ref, cc0_ref, cc1_ref)
    mw = mean(sw0_ref, sw1_ref, cw0_ref, cw1_ref)
    mb = mean(sb0_ref, sb1_ref, cb0_ref, cb1_ref)
    outp_ref[...] = (
        jnp.dot(mc, wlc_ref[...], preferred_element_type=f32)
        + jnp.dot(mw, wlw_ref[...], preferred_element_type=f32)
        + jnp.dot(xp_ref[...], wrc_ref[...] + wrw_ref[...],
                  preferred_element_type=f32)
        + blc_ref[...] + brc_ref[...] + blw_ref[...] + brw_ref[...]
    )
    outa_ref[...] = (
        jnp.dot(mb, wlb_ref[...], preferred_element_type=f32)
        + jnp.dot(xa_ref[...], wrb_ref[...], preferred_element_type=f32)
        + blb_ref[...] + brb_ref[...]
    )


def _tc_epilogue(feats, cnts, xp, xa, ws, bs):
    f32 = jnp.float32
    bm = 1000
    grid = (N_NODE // bm,)
    feat = pl.BlockSpec((bm, D), lambda i: (i, 0))
    cnt = pl.BlockSpec((bm, 1), lambda i: (i, 0))
    wmat = pl.BlockSpec((D, D), lambda i: (0, 0))
    bvec = pl.BlockSpec((1, D), lambda i: (0, 0))
    sc0, sw0, sb0, sc1, sw1, sb1 = feats
    cc0, cw0, cb0, cc1, cw1, cb1 = cnts
    return pl.pallas_call(
        _tc_epilogue_body,
        grid=grid,
        in_specs=[feat, feat, feat, feat, feat, feat,
                  cnt, cnt, cnt, cnt, cnt, cnt, feat, feat,
                  wmat, wmat, wmat, wmat, wmat, wmat,
                  bvec, bvec, bvec, bvec, bvec, bvec],
        out_specs=[feat, feat],
        out_shape=[jax.ShapeDtypeStruct((N_NODE, D), f32),
                   jax.ShapeDtypeStruct((N_NODE, D), f32)],
    )(sc0, sc1, sw0, sw1, sb0, sb1, cc0, cc1, cw0, cw1, cb0, cb1,
      xp, xa, *ws, *bs)


def _unpack_counts(cnt_out):
    res = []
    for cpart in (cnt_out[:CROWS], cnt_out[CROWS:]):
        res.append(cpart.reshape(CROWS * D)[:N_NODE].reshape(N_NODE, 1))
    return res


def kernel(x_paper, x_author, edge_index_cites, edge_index_writes,
           edge_index_written_by,
           Wl_cites, bl_cites, Wr_cites, br_cites,
           Wl_writes, bl_writes, Wr_writes, br_writes,
           Wl_wb, bl_wb, Wr_wb, br_wb):
    i32 = jnp.int32
    f32 = jnp.float32
    ec = edge_index_cites.astype(i32)
    ew = edge_index_writes.astype(i32)
    eb = edge_index_written_by.astype(i32)
    src_c, dst_c = _pad_edges(ec[0], ec[1], CHUNKS_CITES)
    src_w, dst_w = _pad_edges(ew[0], ew[1], CHUNKS_WRITES)
    src_b, dst_b = _pad_edges(eb[0], eb[1], CHUNKS_WRITES)

    zfeat = jnp.zeros((CHUNK, D), f32)
    ident = jnp.eye(D, dtype=f32)

    sum_c, cnt_c, sum_w, cnt_w, sum_b, cnt_b = _sc_segment_sums(
        src_c, dst_c, src_w, dst_w, src_b, dst_b, x_paper, x_author,
        zfeat, ident)

    feats = (sum_c[:N_NODE], sum_w[:N_NODE], sum_b[:N_NODE],
             sum_c[PAD_N:PAD_N + N_NODE], sum_w[PAD_N:PAD_N + N_NODE],
             sum_b[PAD_N:PAD_N + N_NODE])
    cc0, cc1 = _unpack_counts(cnt_c)
    cw0, cw1 = _unpack_counts(cnt_w)
    cb0, cb1 = _unpack_counts(cnt_b)
    cnts = (cc0, cw0, cb0, cc1, cw1, cb1)
    ws = (Wl_cites, Wr_cites, Wl_writes, Wr_writes, Wl_wb, Wr_wb)
    bs = (bl_cites.reshape(1, D), br_cites.reshape(1, D),
          bl_writes.reshape(1, D), br_writes.reshape(1, D),
          bl_wb.reshape(1, D), br_wb.reshape(1, D))
    out_p, out_a = _tc_epilogue(feats, cnts, x_paper, x_author, ws, bs)
    return (out_p, out_a)
